# baseline (speedup 1.0000x reference)
import functools
import math

import jax
import jax.numpy as jnp
from jax.experimental import pallas as pl
from jax.experimental.pallas import tpu as pltpu

D = 2048
H = 16
NOPE = 128
ROPE = 32
VH = 128
KV = 512
E = 8
TK = 2
MI = 512
SH = 2
INTER = 4096
VOC = 32000
L = 3
NDENSE = 1
B = 1
S = 2048
EPS = 1e-6

_F32 = jnp.float32


def _embed_kernel(ids_ref, emb_ref, o_ref, sem, *, bm):
    i = pl.program_id(0)

    def start(r, _):
        idx = ids_ref[i * bm + r]
        pltpu.make_async_copy(emb_ref.at[idx], o_ref.at[r], sem).start()
        return 0

    jax.lax.fori_loop(0, bm, start, 0)

    def wait(r, _):
        pltpu.make_async_copy(emb_ref.at[0], o_ref.at[0], sem).wait()
        return 0

    jax.lax.fori_loop(0, bm, wait, 0)


def _embed(ids, emb, bm=256):
    T = ids.shape[0]
    return pl.pallas_call(
        functools.partial(_embed_kernel, bm=bm),
        grid_spec=pltpu.PrefetchScalarGridSpec(
            num_scalar_prefetch=1,
            grid=(T // bm,),
            in_specs=[pl.BlockSpec(memory_space=pltpu.MemorySpace.HBM)],
            out_specs=pl.BlockSpec((bm, D), lambda i, ids_ref: (i, 0)),
            scratch_shapes=[pltpu.SemaphoreType.DMA],
        ),
        out_shape=jax.ShapeDtypeStruct((T, D), _F32),
    )(ids, emb)


def _rms_kernel(x_ref, g_ref, o_ref):
    x = x_ref[...]
    o_ref[...] = x * jax.lax.rsqrt(jnp.mean(x * x, axis=-1, keepdims=True) + EPS) * g_ref[...]


def _rms(x, g, bm=256):
    M, K = x.shape
    return pl.pallas_call(
        _rms_kernel,
        grid=(M // bm,),
        in_specs=[
            pl.BlockSpec((bm, K), lambda i: (i, 0)),
            pl.BlockSpec((1, K), lambda i: (0, 0)),
        ],
        out_specs=pl.BlockSpec((bm, K), lambda i: (i, 0)),
        out_shape=jax.ShapeDtypeStruct((M, K), _F32),
    )(x, g.reshape(1, K))


def _mm_kernel(x_ref, w_ref, o_ref):
    o_ref[...] = jnp.dot(x_ref[...], w_ref[...], preferred_element_type=_F32)


def _mm(x, w, bm=2048, bn=512):
    M, K = x.shape
    _, N = w.shape
    if K > 2048:
        bm = min(bm, 1024)
    if N % bn != 0:
        bn = N
    return pl.pallas_call(
        _mm_kernel,
        grid=(M // bm, N // bn),
        in_specs=[
            pl.BlockSpec((bm, K), lambda i, j: (i, 0)),
            pl.BlockSpec((K, bn), lambda i, j: (0, j)),
        ],
        out_specs=pl.BlockSpec((bm, bn), lambda i, j: (i, j)),
        out_shape=jax.ShapeDtypeStruct((M, N), _F32),
    )(x, w)


def _rope_kernel(x1_ref, x2_ref, c_ref, s_ref, o1_ref, o2_ref):
    c = c_ref[...]
    s = s_ref[...]
    x1 = x1_ref[0]
    x2 = x2_ref[0]
    o1_ref[0] = x1 * c - x2 * s
    o2_ref[0] = x1 * s + x2 * c


def _rope(x1, x2, cos, sin, bs=512):
    nh, T, half = x1.shape
    return pl.pallas_call(
        _rope_kernel,
        grid=(nh, T // bs),
        in_specs=[
            pl.BlockSpec((1, bs, half), lambda h, i: (h, i, 0)),
            pl.BlockSpec((1, bs, half), lambda h, i: (h, i, 0)),
            pl.BlockSpec((bs, half), lambda h, i: (i, 0)),
            pl.BlockSpec((bs, half), lambda h, i: (i, 0)),
        ],
        out_specs=[
            pl.BlockSpec((1, bs, half), lambda h, i: (h, i, 0)),
            pl.BlockSpec((1, bs, half), lambda h, i: (h, i, 0)),
        ],
        out_shape=[
            jax.ShapeDtypeStruct((nh, T, half), _F32),
            jax.ShapeDtypeStruct((nh, T, half), _F32),
        ],
    )(x1, x2, cos, sin)


def _flash_kernel(qn_ref, qp_ref, kn_ref, kp_ref, v_ref, o_ref, s_ref, *, bq, bk):
    i = pl.program_id(1)
    qn = qn_ref[0]
    qp = qp_ref[0]
    scale = math.sqrt(float(NOPE + ROPE))
    nblk = (i + 1) * bq // bk
    dn = (((1,), (1,)), ((), ()))

    def fill(j, _):
        kn = kn_ref[0, pl.ds(j * bk, bk), :]
        kp = kp_ref[pl.ds(j * bk, bk), :]
        s = jax.lax.dot_general(qn, kn, dn, preferred_element_type=_F32)
        s = s + jax.lax.dot_general(qp, kp, dn, preferred_element_type=_F32)
        s = s / scale
        row = i * bq + jax.lax.broadcasted_iota(jnp.int32, (bq, bk), 0)
        col = j * bk + jax.lax.broadcasted_iota(jnp.int32, (bq, bk), 1)
        s_ref[:, pl.ds(j * bk, bk)] = jnp.where(col <= row, s, -1e30)
        return 0

    jax.lax.fori_loop(0, nblk, fill, 0)

    nb_total = s_ref.shape[1] // bk

    def blank(j, _):
        s_ref[:, pl.ds(j * bk, bk)] = jnp.full((bq, bk), -1e30, _F32)
        return 0

    jax.lax.fori_loop(nblk, nb_total, blank, 0)

    s = s_ref[...]
    m = jnp.max(s, axis=-1, keepdims=True)
    p_un = jnp.exp(s - m)
    l = jnp.sum(p_un, axis=-1, keepdims=True)
    s_ref[...] = p_un / l

    def av(j, acc):
        pj = s_ref[:, pl.ds(j * bk, bk)]
        vj = v_ref[0, pl.ds(j * bk, bk), :]
        return acc + jnp.dot(pj, vj, preferred_element_type=_F32)

    o_ref[0] = jax.lax.fori_loop(0, nblk, av, jnp.zeros((bq, VH), _F32))


def _flash(qn, qp, kn, kp, v, bq=256, bk=256):
    nh, T, _ = qn.shape
    kfn = functools.partial(_flash_kernel, bq=bq, bk=bk)
    return pl.pallas_call(
        kfn,
        grid=(nh, T // bq),
        in_specs=[
            pl.BlockSpec((1, bq, NOPE), lambda h, i: (h, i, 0)),
            pl.BlockSpec((1, bq, ROPE), lambda h, i: (h, i, 0)),
            pl.BlockSpec((1, T, NOPE), lambda h, i: (h, 0, 0)),
            pl.BlockSpec((T, ROPE), lambda h, i: (0, 0)),
            pl.BlockSpec((1, T, VH), lambda h, i: (h, 0, 0)),
        ],
        out_specs=pl.BlockSpec((1, bq, VH), lambda h, i: (h, i, 0)),
        out_shape=jax.ShapeDtypeStruct((nh, T, VH), _F32),
        scratch_shapes=[pltpu.VMEM((bq, T), _F32)],
    )(qn, qp, kn, kp, v)


def _swiglu_kernel(x_ref, w1_ref, w3_ref, o_ref):
    x = x_ref[...]
    a = jnp.dot(x, w1_ref[...], preferred_element_type=_F32)
    b = jnp.dot(x, w3_ref[...], preferred_element_type=_F32)
    o_ref[...] = a * jax.nn.sigmoid(a) * b


def _swiglu_w_kernel(x_ref, w1_ref, w3_ref, c_ref, o_ref):
    x = x_ref[...]
    a = jnp.dot(x, w1_ref[...], preferred_element_type=_F32)
    b = jnp.dot(x, w3_ref[...], preferred_element_type=_F32)
    o_ref[...] = a * jax.nn.sigmoid(a) * b * c_ref[...]


def _swiglu(x, w1, w3, c=None, bm=2048, bn=512):
    M, K = x.shape
    _, N = w1.shape
    if N % bn != 0:
        bn = N
    in_specs = [
        pl.BlockSpec((bm, K), lambda i, j: (i, 0)),
        pl.BlockSpec((K, bn), lambda i, j: (0, j)),
        pl.BlockSpec((K, bn), lambda i, j: (0, j)),
    ]
    args = [x, w1, w3]
    kfn = _swiglu_kernel
    if c is not None:
        in_specs.append(pl.BlockSpec((bm, 1), lambda i, j: (i, 0)))
        args.append(c)
        kfn = _swiglu_w_kernel
    return pl.pallas_call(
        kfn,
        grid=(M // bm, N // bn),
        in_specs=in_specs,
        out_specs=pl.BlockSpec((bm, bn), lambda i, j: (i, j)),
        out_shape=jax.ShapeDtypeStruct((M, N), _F32),
    )(*args)


def _router_kernel(x_ref, g_ref, o_ref):
    s = jnp.dot(x_ref[...], g_ref[...], preferred_element_type=_F32)
    s = jax.nn.softmax(s, axis=-1)
    iota = jax.lax.broadcasted_iota(jnp.int32, s.shape, 1)
    i1 = jnp.argmax(s, axis=-1)
    oh1 = iota == i1[:, None]
    m1 = jnp.max(s, axis=-1, keepdims=True)
    s2 = jnp.where(oh1, -jnp.inf, s)
    i2 = jnp.argmax(s2, axis=-1)
    oh2 = iota == i2[:, None]
    m2 = jnp.max(s2, axis=-1, keepdims=True)
    o_ref[...] = jnp.where(oh1, m1, 0.0) + jnp.where(oh2, m2, 0.0)


def _router(x, gate, bm=256):
    M, K = x.shape
    return pl.pallas_call(
        _router_kernel,
        grid=(M // bm,),
        in_specs=[
            pl.BlockSpec((bm, K), lambda i: (i, 0)),
            pl.BlockSpec((K, E), lambda i: (0, 0)),
        ],
        out_specs=pl.BlockSpec((bm, E), lambda i: (i, 0)),
        out_shape=jax.ShapeDtypeStruct((M, E), _F32),
    )(x, gate)


def _rope_tables():
    half = ROPE // 2
    pos = jnp.arange(S, dtype=jnp.float32)
    inv = 1.0 / (10000.0 ** (jnp.arange(half, dtype=jnp.float32) / half))
    ang = pos[:, None] * inv[None, :]
    return jnp.cos(ang), jnp.sin(ang)


def _attn_layer(p, x, cos, sin):
    h = _rms(x, p['attn_norm'])
    q = _mm(h, p['wq'])
    kv = _mm(h, p['wkv_a'])
    kv_c = _rms(kv[:, :KV], p['kv_norm'])
    kvb = _mm(kv_c, p['wkv_b'])

    q3 = q.reshape(S, H, NOPE + ROPE)
    qn = q3[..., :NOPE].transpose(1, 0, 2)
    qp_half = ROPE // 2
    q_pe = q3[..., NOPE:]
    qp1 = q_pe[..., :qp_half].transpose(1, 0, 2)
    qp2 = q_pe[..., qp_half:].transpose(1, 0, 2)
    qp1r, qp2r = _rope(qp1, qp2, cos, sin)
    qp = jnp.concatenate([qp1r, qp2r], axis=-1)

    k_pe = kv[:, KV:]
    kp1 = k_pe[:, :qp_half][None]
    kp2 = k_pe[:, qp_half:][None]
    kp1r, kp2r = _rope(kp1, kp2, cos, sin)
    kp = jnp.concatenate([kp1r, kp2r], axis=-1)[0]

    kvb3 = kvb.reshape(S, H, NOPE + VH)
    kn = kvb3[..., :NOPE].transpose(1, 0, 2)
    v = kvb3[..., NOPE:].transpose(1, 0, 2)

    o = _flash(qn, qp, kn, kp, v)
    o2 = o.transpose(1, 0, 2).reshape(S, H * VH)
    return _mm(o2, p['wo'])


_TILE = 128
_P = 2 * S + E * _TILE


def _expert_kernel(tok_ref, ext_ref, hh_ref, w1_ref, w3_ref, w2_ref, c_ref,
                   o_ref, x_scr, sem):
    i = pl.program_id(0)

    def start(r, _):
        idx = tok_ref[i * _TILE + r]
        pltpu.make_async_copy(hh_ref.at[idx], x_scr.at[r], sem).start()
        return 0

    jax.lax.fori_loop(0, _TILE, start, 0)

    def wait(r, _):
        pltpu.make_async_copy(hh_ref.at[0], x_scr.at[0], sem).wait()
        return 0

    jax.lax.fori_loop(0, _TILE, wait, 0)

    x = x_scr[...]
    a = jnp.dot(x, w1_ref[0], preferred_element_type=_F32)
    b = jnp.dot(x, w3_ref[0], preferred_element_type=_F32)
    g = a * jax.nn.sigmoid(a) * b
    y = jnp.dot(g, w2_ref[0], preferred_element_type=_F32)
    o_ref[...] = y * c_ref[...]


def _moe_combine_kernel(s1_ref, s2_ref, y_ref, sh_ref, o_ref,
                        y1_scr, y2_scr, sem1, sem2, *, bm):
    i = pl.program_id(0)

    def start(r, _):
        pltpu.make_async_copy(y_ref.at[s1_ref[i * bm + r]], y1_scr.at[r], sem1).start()
        pltpu.make_async_copy(y_ref.at[s2_ref[i * bm + r]], y2_scr.at[r], sem2).start()
        return 0

    jax.lax.fori_loop(0, bm, start, 0)

    def wait(r, _):
        pltpu.make_async_copy(y_ref.at[0], y1_scr.at[0], sem1).wait()
        pltpu.make_async_copy(y_ref.at[0], y2_scr.at[0], sem2).wait()
        return 0

    jax.lax.fori_loop(0, bm, wait, 0)
    o_ref[...] = (y1_scr[...] + y2_scr[...]) + sh_ref[...]


def _moe_layer(p, hh):
    comb = _router(hh, p['gate'])

    mask = comb > 0.0
    counts = jnp.sum(mask.astype(jnp.int32), axis=0)
    padded = ((counts + _TILE - 1) // _TILE) * _TILE
    seg_off = jnp.concatenate([jnp.zeros((1,), jnp.int32),
                               jnp.cumsum(padded)[:-1].astype(jnp.int32)])
    seg_end = jnp.cumsum(padded).astype(jnp.int32)
    rank = jnp.cumsum(mask.astype(jnp.int32), axis=0) - mask.astype(jnp.int32)
    slot = seg_off[None, :] + rank
    slot_full = jnp.where(mask, slot, _P).astype(jnp.int32).reshape(-1)
    tt = jnp.broadcast_to(jnp.arange(S, dtype=jnp.int32)[:, None], (S, E)).reshape(-1)
    tok_src = jnp.zeros((_P + 1,), jnp.int32).at[slot_full].set(tt, mode='drop')[:_P]
    wgt = jnp.zeros((_P + 1,), _F32).at[slot_full].set(comb.reshape(-1), mode='drop')[:_P]
    tile_start = jnp.arange(_P // _TILE, dtype=jnp.int32) * _TILE
    ex_tile = jnp.minimum(jnp.searchsorted(seg_end, tile_start, side='right'),
                          E - 1).astype(jnp.int32)
    e1 = jnp.argmax(mask, axis=1)
    e2 = E - 1 - jnp.argmax(mask[:, ::-1], axis=1)
    tr = jnp.arange(S)
    s1 = slot[tr, e1].astype(jnp.int32)
    s2 = slot[tr, e2].astype(jnp.int32)

    y = pl.pallas_call(
        _expert_kernel,
        grid_spec=pltpu.PrefetchScalarGridSpec(
            num_scalar_prefetch=2,
            grid=(_P // _TILE,),
            in_specs=[
                pl.BlockSpec(memory_space=pltpu.MemorySpace.HBM),
                pl.BlockSpec((1, D, MI), lambda i, tok, ext: (ext[i], 0, 0)),
                pl.BlockSpec((1, D, MI), lambda i, tok, ext: (ext[i], 0, 0)),
                pl.BlockSpec((1, MI, D), lambda i, tok, ext: (ext[i], 0, 0)),
                pl.BlockSpec((_TILE, 1), lambda i, tok, ext: (i, 0)),
            ],
            out_specs=pl.BlockSpec((_TILE, D), lambda i, tok, ext: (i, 0)),
            scratch_shapes=[pltpu.VMEM((_TILE, D), _F32), pltpu.SemaphoreType.DMA],
        ),
        out_shape=jax.ShapeDtypeStruct((_P, D), _F32),
    )(tok_src, ex_tile, hh, p['ew1'], p['ew3'], p['ew2'], wgt.reshape(_P, 1))

    sh = _mm(_swiglu(hh, p['sw1'], p['sw3']), p['sw2'])

    bm = 256
    return pl.pallas_call(
        functools.partial(_moe_combine_kernel, bm=bm),
        grid_spec=pltpu.PrefetchScalarGridSpec(
            num_scalar_prefetch=2,
            grid=(S // bm,),
            in_specs=[
                pl.BlockSpec(memory_space=pltpu.MemorySpace.HBM),
                pl.BlockSpec((bm, D), lambda i, s1, s2: (i, 0)),
            ],
            out_specs=pl.BlockSpec((bm, D), lambda i, s1, s2: (i, 0)),
            scratch_shapes=[pltpu.VMEM((bm, D), _F32), pltpu.VMEM((bm, D), _F32),
                            pltpu.SemaphoreType.DMA, pltpu.SemaphoreType.DMA],
        ),
        out_shape=jax.ShapeDtypeStruct((S, D), _F32),
    )(s1, s2, y, sh)


def kernel(input_ids, params):
    ids = input_ids.reshape(B * S)
    x = _embed(ids, params['embed'])
    cos, sin = _rope_tables()
    for i in range(L):
        p = params['layer_%d' % i]
        x = x + _attn_layer(p, x, cos, sin)
        hh = _rms(x, p['ffn_norm'])
        if i < NDENSE:
            g = _swiglu(hh, p['w1'], p['w3'])
            x = x + _mm(g, p['w2'])
        else:
            x = x + _moe_layer(p, hh)
    h = _rms(x, params['final_norm'])
    logits = _mm(h, params['head'], bm=2048, bn=640)
    return logits.reshape(B, S, VOC)


# speedup vs baseline: 1.0709x; 1.0709x over previous
import functools
import math

import jax
import jax.numpy as jnp
from jax import lax
from jax.experimental import pallas as pl
from jax.experimental.pallas import tpu as pltpu
from jax.experimental.pallas import tpu_sc as plsc

D = 2048
H = 16
NOPE = 128
ROPE = 32
VH = 128
KV = 512
E = 8
TK = 2
MI = 512
SH = 2
INTER = 4096
VOC = 32000
L = 3
NDENSE = 1
B = 1
S = 2048
EPS = 1e-6

_F32 = jnp.float32


def _sc_gather(table, idx, chunk=32):
    V, Dd = table.shape
    Bn = idx.shape[0]
    info = plsc.get_sparse_core_info()
    NW = info.num_cores * info.num_subcores
    b_per_w = Bn // NW
    n_chunks = b_per_w // chunk
    mesh = plsc.VectorSubcoreMesh(core_axis_name="c", subcore_axis_name="s")

    def body(table_hbm, idx_hbm, out_hbm, idx_v, rows_v, sem):
        wid = lax.axis_index("s") * info.num_cores + lax.axis_index("c")
        base = wid * b_per_w
        for c in range(n_chunks):
            off = base + c * chunk
            pltpu.sync_copy(idx_hbm.at[pl.ds(off, chunk)], idx_v)
            pltpu.async_copy(table_hbm.at[idx_v], rows_v, sem).wait()
            pltpu.sync_copy(rows_v, out_hbm.at[pl.ds(off, chunk)])

    return pl.kernel(
        body,
        out_type=jax.ShapeDtypeStruct((Bn, Dd), table.dtype),
        mesh=mesh,
        scratch_types=[
            pltpu.VMEM((chunk,), jnp.int32),
            pltpu.VMEM((chunk, Dd), table.dtype),
            pltpu.SemaphoreType.DMA,
        ],
    )(table, idx)


def _embed_kernel(ids_ref, emb_ref, o_ref, sem, *, bm):
    i = pl.program_id(0)

    def start(r, _):
        idx = ids_ref[i * bm + r]
        pltpu.make_async_copy(emb_ref.at[idx], o_ref.at[r], sem).start()
        return 0

    jax.lax.fori_loop(0, bm, start, 0)

    def wait(r, _):
        pltpu.make_async_copy(emb_ref.at[0], o_ref.at[0], sem).wait()
        return 0

    jax.lax.fori_loop(0, bm, wait, 0)


def _embed(ids, emb, bm=256):
    T = ids.shape[0]
    return pl.pallas_call(
        functools.partial(_embed_kernel, bm=bm),
        grid_spec=pltpu.PrefetchScalarGridSpec(
            num_scalar_prefetch=1,
            grid=(T // bm,),
            in_specs=[pl.BlockSpec(memory_space=pltpu.MemorySpace.HBM)],
            out_specs=pl.BlockSpec((bm, D), lambda i, ids_ref: (i, 0)),
            scratch_shapes=[pltpu.SemaphoreType.DMA],
        ),
        out_shape=jax.ShapeDtypeStruct((T, D), _F32),
    )(ids, emb)


def _rms_kernel(x_ref, g_ref, o_ref):
    x = x_ref[...]
    o_ref[...] = x * jax.lax.rsqrt(jnp.mean(x * x, axis=-1, keepdims=True) + EPS) * g_ref[...]


def _rms(x, g, bm=256):
    M, K = x.shape
    return pl.pallas_call(
        _rms_kernel,
        grid=(M // bm,),
        in_specs=[
            pl.BlockSpec((bm, K), lambda i: (i, 0)),
            pl.BlockSpec((1, K), lambda i: (0, 0)),
        ],
        out_specs=pl.BlockSpec((bm, K), lambda i: (i, 0)),
        out_shape=jax.ShapeDtypeStruct((M, K), _F32),
    )(x, g.reshape(1, K))


def _mm_kernel(x_ref, w_ref, o_ref):
    o_ref[...] = jnp.dot(x_ref[...], w_ref[...], preferred_element_type=_F32)


def _mm(x, w, bm=2048, bn=512):
    M, K = x.shape
    _, N = w.shape
    if K > 2048:
        bm = min(bm, 1024)
    if N % bn != 0:
        bn = N
    return pl.pallas_call(
        _mm_kernel,
        grid=(M // bm, N // bn),
        in_specs=[
            pl.BlockSpec((bm, K), lambda i, j: (i, 0)),
            pl.BlockSpec((K, bn), lambda i, j: (0, j)),
        ],
        out_specs=pl.BlockSpec((bm, bn), lambda i, j: (i, j)),
        out_shape=jax.ShapeDtypeStruct((M, N), _F32),
    )(x, w)


def _rope_kernel(x1_ref, x2_ref, c_ref, s_ref, o1_ref, o2_ref):
    c = c_ref[...]
    s = s_ref[...]
    x1 = x1_ref[0]
    x2 = x2_ref[0]
    o1_ref[0] = x1 * c - x2 * s
    o2_ref[0] = x1 * s + x2 * c


def _rope(x1, x2, cos, sin, bs=512):
    nh, T, half = x1.shape
    return pl.pallas_call(
        _rope_kernel,
        grid=(nh, T // bs),
        in_specs=[
            pl.BlockSpec((1, bs, half), lambda h, i: (h, i, 0)),
            pl.BlockSpec((1, bs, half), lambda h, i: (h, i, 0)),
            pl.BlockSpec((bs, half), lambda h, i: (i, 0)),
            pl.BlockSpec((bs, half), lambda h, i: (i, 0)),
        ],
        out_specs=[
            pl.BlockSpec((1, bs, half), lambda h, i: (h, i, 0)),
            pl.BlockSpec((1, bs, half), lambda h, i: (h, i, 0)),
        ],
        out_shape=[
            jax.ShapeDtypeStruct((nh, T, half), _F32),
            jax.ShapeDtypeStruct((nh, T, half), _F32),
        ],
    )(x1, x2, cos, sin)


def _flash_kernel(qn_ref, qp_ref, kn_ref, kp_ref, v_ref, o_ref, s_ref, *, bq, bk):
    i = pl.program_id(1)
    qn = qn_ref[0]
    qp = qp_ref[0]
    scale = math.sqrt(float(NOPE + ROPE))
    nblk = (i + 1) * bq // bk
    dn = (((1,), (1,)), ((), ()))

    def fill(j, _):
        kn = kn_ref[0, pl.ds(j * bk, bk), :]
        kp = kp_ref[pl.ds(j * bk, bk), :]
        s = jax.lax.dot_general(qn, kn, dn, preferred_element_type=_F32)
        s = s + jax.lax.dot_general(qp, kp, dn, preferred_element_type=_F32)
        s = s / scale
        row = i * bq + jax.lax.broadcasted_iota(jnp.int32, (bq, bk), 0)
        col = j * bk + jax.lax.broadcasted_iota(jnp.int32, (bq, bk), 1)
        s_ref[:, pl.ds(j * bk, bk)] = jnp.where(col <= row, s, -1e30)
        return 0

    jax.lax.fori_loop(0, nblk, fill, 0)

    nb_total = s_ref.shape[1] // bk

    def blank(j, _):
        s_ref[:, pl.ds(j * bk, bk)] = jnp.full((bq, bk), -1e30, _F32)
        return 0

    jax.lax.fori_loop(nblk, nb_total, blank, 0)

    s = s_ref[...]
    m = jnp.max(s, axis=-1, keepdims=True)
    p_un = jnp.exp(s - m)
    l = jnp.sum(p_un, axis=-1, keepdims=True)
    s_ref[...] = p_un / l

    def av(j, acc):
        pj = s_ref[:, pl.ds(j * bk, bk)]
        vj = v_ref[0, pl.ds(j * bk, bk), :]
        return acc + jnp.dot(pj, vj, preferred_element_type=_F32)

    o_ref[0] = jax.lax.fori_loop(0, nblk, av, jnp.zeros((bq, VH), _F32))


def _flash(qn, qp, kn, kp, v, bq=256, bk=256):
    nh, T, _ = qn.shape
    kfn = functools.partial(_flash_kernel, bq=bq, bk=bk)
    return pl.pallas_call(
        kfn,
        grid=(nh, T // bq),
        in_specs=[
            pl.BlockSpec((1, bq, NOPE), lambda h, i: (h, i, 0)),
            pl.BlockSpec((1, bq, ROPE), lambda h, i: (h, i, 0)),
            pl.BlockSpec((1, T, NOPE), lambda h, i: (h, 0, 0)),
            pl.BlockSpec((T, ROPE), lambda h, i: (0, 0)),
            pl.BlockSpec((1, T, VH), lambda h, i: (h, 0, 0)),
        ],
        out_specs=pl.BlockSpec((1, bq, VH), lambda h, i: (h, i, 0)),
        out_shape=jax.ShapeDtypeStruct((nh, T, VH), _F32),
        scratch_shapes=[pltpu.VMEM((bq, T), _F32)],
    )(qn, qp, kn, kp, v)


def _swiglu_kernel(x_ref, w1_ref, w3_ref, o_ref):
    x = x_ref[...]
    a = jnp.dot(x, w1_ref[...], preferred_element_type=_F32)
    b = jnp.dot(x, w3_ref[...], preferred_element_type=_F32)
    o_ref[...] = a * jax.nn.sigmoid(a) * b


def _swiglu_w_kernel(x_ref, w1_ref, w3_ref, c_ref, o_ref):
    x = x_ref[...]
    a = jnp.dot(x, w1_ref[...], preferred_element_type=_F32)
    b = jnp.dot(x, w3_ref[...], preferred_element_type=_F32)
    o_ref[...] = a * jax.nn.sigmoid(a) * b * c_ref[...]


def _swiglu(x, w1, w3, c=None, bm=2048, bn=512):
    M, K = x.shape
    _, N = w1.shape
    if N % bn != 0:
        bn = N
    in_specs = [
        pl.BlockSpec((bm, K), lambda i, j: (i, 0)),
        pl.BlockSpec((K, bn), lambda i, j: (0, j)),
        pl.BlockSpec((K, bn), lambda i, j: (0, j)),
    ]
    args = [x, w1, w3]
    kfn = _swiglu_kernel
    if c is not None:
        in_specs.append(pl.BlockSpec((bm, 1), lambda i, j: (i, 0)))
        args.append(c)
        kfn = _swiglu_w_kernel
    return pl.pallas_call(
        kfn,
        grid=(M // bm, N // bn),
        in_specs=in_specs,
        out_specs=pl.BlockSpec((bm, bn), lambda i, j: (i, j)),
        out_shape=jax.ShapeDtypeStruct((M, N), _F32),
    )(*args)


def _router_kernel(x_ref, g_ref, o_ref):
    s = jnp.dot(x_ref[...], g_ref[...], preferred_element_type=_F32)
    s = jax.nn.softmax(s, axis=-1)
    iota = jax.lax.broadcasted_iota(jnp.int32, s.shape, 1)
    i1 = jnp.argmax(s, axis=-1)
    oh1 = iota == i1[:, None]
    m1 = jnp.max(s, axis=-1, keepdims=True)
    s2 = jnp.where(oh1, -jnp.inf, s)
    i2 = jnp.argmax(s2, axis=-1)
    oh2 = iota == i2[:, None]
    m2 = jnp.max(s2, axis=-1, keepdims=True)
    o_ref[...] = jnp.where(oh1, m1, 0.0) + jnp.where(oh2, m2, 0.0)


def _router(x, gate, bm=256):
    M, K = x.shape
    return pl.pallas_call(
        _router_kernel,
        grid=(M // bm,),
        in_specs=[
            pl.BlockSpec((bm, K), lambda i: (i, 0)),
            pl.BlockSpec((K, E), lambda i: (0, 0)),
        ],
        out_specs=pl.BlockSpec((bm, E), lambda i: (i, 0)),
        out_shape=jax.ShapeDtypeStruct((M, E), _F32),
    )(x, gate)


def _rope_tables():
    half = ROPE // 2
    pos = jnp.arange(S, dtype=jnp.float32)
    inv = 1.0 / (10000.0 ** (jnp.arange(half, dtype=jnp.float32) / half))
    ang = pos[:, None] * inv[None, :]
    return jnp.cos(ang), jnp.sin(ang)


def _attn_layer(p, x, cos, sin):
    h = _rms(x, p['attn_norm'])
    q = _mm(h, p['wq'])
    kv = _mm(h, p['wkv_a'])
    kv_c = _rms(kv[:, :KV], p['kv_norm'])
    kvb = _mm(kv_c, p['wkv_b'])

    q3 = q.reshape(S, H, NOPE + ROPE)
    qn = q3[..., :NOPE].transpose(1, 0, 2)
    qp_half = ROPE // 2
    q_pe = q3[..., NOPE:]
    qp1 = q_pe[..., :qp_half].transpose(1, 0, 2)
    qp2 = q_pe[..., qp_half:].transpose(1, 0, 2)
    qp1r, qp2r = _rope(qp1, qp2, cos, sin)
    qp = jnp.concatenate([qp1r, qp2r], axis=-1)

    k_pe = kv[:, KV:]
    kp1 = k_pe[:, :qp_half][None]
    kp2 = k_pe[:, qp_half:][None]
    kp1r, kp2r = _rope(kp1, kp2, cos, sin)
    kp = jnp.concatenate([kp1r, kp2r], axis=-1)[0]

    kvb3 = kvb.reshape(S, H, NOPE + VH)
    kn = kvb3[..., :NOPE].transpose(1, 0, 2)
    v = kvb3[..., NOPE:].transpose(1, 0, 2)

    o = _flash(qn, qp, kn, kp, v)
    o2 = o.transpose(1, 0, 2).reshape(S, H * VH)
    return _mm(o2, p['wo'])


_TILE = 128
_P = 2 * S + E * _TILE


def _expert_kernel(ext_ref, x_ref, w1_ref, w3_ref, w2_ref, c_ref, o_ref):
    x = x_ref[...]
    a = jnp.dot(x, w1_ref[0], preferred_element_type=_F32)
    b = jnp.dot(x, w3_ref[0], preferred_element_type=_F32)
    g = a * jax.nn.sigmoid(a) * b
    y = jnp.dot(g, w2_ref[0], preferred_element_type=_F32)
    o_ref[...] = y * c_ref[...]


def _moe_combine_kernel(y1_ref, y2_ref, sh_ref, o_ref):
    o_ref[...] = (y1_ref[...] + y2_ref[...]) + sh_ref[...]


def _moe_layer(p, hh):
    comb = _router(hh, p['gate'])

    mask = comb > 0.0
    counts = jnp.sum(mask.astype(jnp.int32), axis=0)
    padded = ((counts + _TILE - 1) // _TILE) * _TILE
    seg_off = jnp.concatenate([jnp.zeros((1,), jnp.int32),
                               jnp.cumsum(padded)[:-1].astype(jnp.int32)])
    seg_end = jnp.cumsum(padded).astype(jnp.int32)
    rank = jnp.cumsum(mask.astype(jnp.int32), axis=0) - mask.astype(jnp.int32)
    slot = seg_off[None, :] + rank
    slot_full = jnp.where(mask, slot, _P).astype(jnp.int32).reshape(-1)
    tt = jnp.broadcast_to(jnp.arange(S, dtype=jnp.int32)[:, None], (S, E)).reshape(-1)
    tok_src = jnp.zeros((_P + 1,), jnp.int32).at[slot_full].set(tt, mode='drop')[:_P]
    wgt = jnp.zeros((_P + 1,), _F32).at[slot_full].set(comb.reshape(-1), mode='drop')[:_P]
    tile_start = jnp.arange(_P // _TILE, dtype=jnp.int32) * _TILE
    ex_tile = jnp.minimum(jnp.searchsorted(seg_end, tile_start, side='right'),
                          E - 1).astype(jnp.int32)
    e1 = jnp.argmax(mask, axis=1)
    e2 = E - 1 - jnp.argmax(mask[:, ::-1], axis=1)
    tr = jnp.arange(S)
    s1 = slot[tr, e1].astype(jnp.int32)
    s2 = slot[tr, e2].astype(jnp.int32)

    xs = _sc_gather(hh, tok_src)

    y = pl.pallas_call(
        _expert_kernel,
        grid_spec=pltpu.PrefetchScalarGridSpec(
            num_scalar_prefetch=1,
            grid=(_P // _TILE,),
            in_specs=[
                pl.BlockSpec((_TILE, D), lambda i, ext: (i, 0)),
                pl.BlockSpec((1, D, MI), lambda i, ext: (ext[i], 0, 0)),
                pl.BlockSpec((1, D, MI), lambda i, ext: (ext[i], 0, 0)),
                pl.BlockSpec((1, MI, D), lambda i, ext: (ext[i], 0, 0)),
                pl.BlockSpec((_TILE, 1), lambda i, ext: (i, 0)),
            ],
            out_specs=pl.BlockSpec((_TILE, D), lambda i, ext: (i, 0)),
        ),
        out_shape=jax.ShapeDtypeStruct((_P, D), _F32),
    )(ex_tile, xs, p['ew1'], p['ew3'], p['ew2'], wgt.reshape(_P, 1))

    sh = _mm(_swiglu(hh, p['sw1'], p['sw3']), p['sw2'])
    y1 = _sc_gather(y, s1)
    y2 = _sc_gather(y, s2)

    bm = 256
    return pl.pallas_call(
        _moe_combine_kernel,
        grid=(S // bm,),
        in_specs=[
            pl.BlockSpec((bm, D), lambda i: (i, 0)),
            pl.BlockSpec((bm, D), lambda i: (i, 0)),
            pl.BlockSpec((bm, D), lambda i: (i, 0)),
        ],
        out_specs=pl.BlockSpec((bm, D), lambda i: (i, 0)),
        out_shape=jax.ShapeDtypeStruct((S, D), _F32),
    )(y1, y2, sh)


def kernel(input_ids, params):
    ids = input_ids.reshape(B * S)
    x = _sc_gather(params['embed'], ids)
    cos, sin = _rope_tables()
    for i in range(L):
        p = params['layer_%d' % i]
        x = x + _attn_layer(p, x, cos, sin)
        hh = _rms(x, p['ffn_norm'])
        if i < NDENSE:
            g = _swiglu(hh, p['w1'], p['w3'])
            x = x + _mm(g, p['w2'])
        else:
            x = x + _moe_layer(p, hh)
    h = _rms(x, params['final_norm'])
    logits = _mm(h, params['head'], bm=2048, bn=640)
    return logits.reshape(B, S, VOC)
